# Initial kernel scaffold; baseline (speedup 1.0000x reference)
import functools

import jax
import jax.numpy as jnp
from jax import lax
from jax.experimental import pallas as pl
from jax.experimental.pallas import tpu as pltpu
from jax.experimental.pallas import tpu_sc as plsc

N = 50000
E = 800000
IN_DIM = 1433
HID = 64
HALF = 32
OUT_DIM = 7
OUT_PAD = 8

NC = 2
NS = 16
N_TILE = N // NS
BM = 400
GRID_M = N // BM

CH_B = 80
NCH_B = E // NS // CH_B
CH_2 = 40
NCH_2 = E // (NC * NS) // CH_2

_mesh = plsc.VectorSubcoreMesh(core_axis_name="c", subcore_axis_name="s")


@functools.partial(
    pl.kernel,
    out_type=jax.ShapeDtypeStruct((2 * N, OUT_PAD), jnp.float32),
    scratch_types=[
        pltpu.VMEM((NCH_2, CH_2), jnp.int32),
        pltpu.VMEM((CH_2, OUT_PAD), jnp.float32),
        pltpu.VMEM_SHARED((N, OUT_PAD), jnp.float32),
    ],
    mesh=_mesh,
)
def _sc_deg(dst_hbm, ones_hbm, z_hbm, out_hbm, dst_v, ones_v, acc):
    c = lax.axis_index("c")
    s = lax.axis_index("s")
    wid = s * NC + c
    pltpu.sync_copy(z_hbm.at[pl.ds(s * N_TILE, N_TILE)],
                    acc.at[pl.ds(s * N_TILE, N_TILE)])
    pltpu.sync_copy(dst_hbm.at[pl.ds(wid * NCH_2, NCH_2)], dst_v)
    pltpu.sync_copy(ones_hbm, ones_v)
    plsc.subcore_barrier()

    def body(j, carry):
        pltpu.sync_copy(ones_v, acc.at[dst_v.at[j]], add=True)
        return carry

    lax.fori_loop(0, NCH_2, body, 0)
    plsc.subcore_barrier()
    pltpu.sync_copy(acc.at[pl.ds(s * N_TILE, N_TILE)],
                    out_hbm.at[pl.ds(c * N + s * N_TILE, N_TILE)])


@functools.partial(
    pl.kernel,
    out_type=jax.ShapeDtypeStruct((2 * N, HALF), jnp.float32),
    scratch_types=[
        pltpu.VMEM((NCH_B, CH_B), jnp.int32),
        pltpu.VMEM((NCH_B, CH_B), jnp.int32),
        pltpu.VMEM((CH_B, HALF), jnp.float32),
        pltpu.VMEM_SHARED((N, HALF), jnp.float32),
        pltpu.SemaphoreType.DMA,
    ],
    mesh=_mesh,
)
def _sc_scatter32(src_hbm, dst_hbm, y0_hbm, y1_hbm, out_hbm,
                  src_v, dst_v, rows_v, acc, sem):
    c = lax.axis_index("c")
    s = lax.axis_index("s")

    def init(y_hbm):
        pltpu.sync_copy(y_hbm.at[pl.ds(s * N_TILE, N_TILE)],
                        acc.at[pl.ds(s * N_TILE, N_TILE)])

    @pl.when(c == 0)
    def _():
        init(y0_hbm)

    @pl.when(c == 1)
    def _():
        init(y1_hbm)

    pltpu.sync_copy(src_hbm.at[pl.ds(s * NCH_B, NCH_B)], src_v)
    pltpu.sync_copy(dst_hbm.at[pl.ds(s * NCH_B, NCH_B)], dst_v)
    plsc.subcore_barrier()

    def edges(y_hbm):
        def body(j, carry):
            pltpu.async_copy(y_hbm.at[src_v.at[j]], rows_v, sem).wait()
            pltpu.sync_copy(rows_v, acc.at[dst_v.at[j]], add=True)
            return carry
        lax.fori_loop(0, NCH_B, body, 0)

    @pl.when(c == 0)
    def _():
        edges(y0_hbm)

    @pl.when(c == 1)
    def _():
        edges(y1_hbm)

    plsc.subcore_barrier()
    pltpu.sync_copy(acc.at[pl.ds(s * N_TILE, N_TILE)],
                    out_hbm.at[pl.ds(c * N + s * N_TILE, N_TILE)])


@functools.partial(
    pl.kernel,
    out_type=jax.ShapeDtypeStruct((2 * N, OUT_PAD), jnp.float32),
    scratch_types=[
        pltpu.VMEM((NCH_2, CH_2), jnp.int32),
        pltpu.VMEM((NCH_2, CH_2), jnp.int32),
        pltpu.VMEM((CH_2, OUT_PAD), jnp.float32),
        pltpu.VMEM_SHARED((N, OUT_PAD), jnp.float32),
        pltpu.SemaphoreType.DMA,
    ],
    mesh=_mesh,
)
def _sc_scatter8(src_hbm, dst_hbm, y_hbm, out_hbm, src_v, dst_v, rows_v, acc, sem):
    c = lax.axis_index("c")
    s = lax.axis_index("s")
    wid = s * NC + c

    pltpu.sync_copy(y_hbm.at[pl.ds(s * N_TILE, N_TILE)],
                    acc.at[pl.ds(s * N_TILE, N_TILE)])
    pltpu.sync_copy(src_hbm.at[pl.ds(wid * NCH_2, NCH_2)], src_v)
    pltpu.sync_copy(dst_hbm.at[pl.ds(wid * NCH_2, NCH_2)], dst_v)
    plsc.subcore_barrier()

    def body(j, carry):
        pltpu.async_copy(y_hbm.at[src_v.at[j]], rows_v, sem).wait()
        pltpu.sync_copy(rows_v, acc.at[dst_v.at[j]], add=True)
        return carry

    lax.fori_loop(0, NCH_2, body, 0)
    plsc.subcore_barrier()
    pltpu.sync_copy(acc.at[pl.ds(s * N_TILE, N_TILE)],
                    out_hbm.at[pl.ds(c * N + s * N_TILE, N_TILE)])


def _dis_block(d0, d1):
    return lax.rsqrt(d0[:, :1] + d1[:, :1] + 1.0)


def _m1_body(x_ref, wa_ref, wb_ref, d0_ref, d1_ref, y0_ref, y1_ref):
    dis = _dis_block(d0_ref[...], d1_ref[...])
    xb = x_ref[...]
    y0_ref[...] = dis * jnp.dot(xb, wa_ref[...], preferred_element_type=jnp.float32)
    y1_ref[...] = dis * jnp.dot(xb, wb_ref[...], preferred_element_type=jnp.float32)


def _m2_body(a0_ref, a1_ref, d0_ref, d1_ref, w2_ref, b1_ref, y2_ref):
    dis = _dis_block(d0_ref[...], d1_ref[...])
    h = jnp.concatenate([a0_ref[...], a1_ref[...]], axis=1)
    h = jnp.maximum(dis * h + b1_ref[...], 0.0)
    y2_ref[...] = dis * jnp.dot(h, w2_ref[...], preferred_element_type=jnp.float32)


def _m3_body(c0_ref, c1_ref, y2_ref, d0_ref, d1_ref, b2_ref, o_ref):
    dis = _dis_block(d0_ref[...], d1_ref[...])
    acc = c0_ref[...] + c1_ref[...] - y2_ref[...]
    o_ref[...] = dis * acc + b2_ref[...]


def _row_spec(w):
    return pl.BlockSpec((BM, w), lambda i: (i, 0))


def _full_spec(h, w):
    return pl.BlockSpec((h, w), lambda i: (0, 0))


def kernel(x, edge_index, W1, b1, W2, b2):
    ei = edge_index.astype(jnp.int32)
    src = ei[0]
    dst = ei[1]
    src_b = src.reshape(E // CH_B, CH_B)
    dst_b = dst.reshape(E // CH_B, CH_B)
    src_2 = src.reshape(E // CH_2, CH_2)
    dst_2 = dst.reshape(E // CH_2, CH_2)
    ones8 = jnp.ones((CH_2, OUT_PAD), jnp.float32)
    zeros8 = jnp.zeros((N, OUT_PAD), jnp.float32)

    degw = _sc_deg(dst_2, ones8, zeros8)
    d0, d1 = degw[:N], degw[N:]

    y0, y1 = pl.pallas_call(
        _m1_body,
        grid=(GRID_M,),
        in_specs=[
            pl.BlockSpec((BM, IN_DIM), lambda i: (i, 0)),
            _full_spec(IN_DIM, HALF),
            _full_spec(IN_DIM, HALF),
            _row_spec(OUT_PAD),
            _row_spec(OUT_PAD),
        ],
        out_specs=[_row_spec(HALF), _row_spec(HALF)],
        out_shape=[
            jax.ShapeDtypeStruct((N, HALF), jnp.float32),
            jax.ShapeDtypeStruct((N, HALF), jnp.float32),
        ],
    )(x, W1[:, :HALF], W1[:, HALF:], d0, d1)

    accw = _sc_scatter32(src_b, dst_b, y0, y1)
    a0, a1 = accw[:N], accw[N:]

    w2p = jnp.zeros((HID, OUT_PAD), jnp.float32).at[:, :OUT_DIM].set(W2)
    b2p = jnp.zeros((1, OUT_PAD), jnp.float32).at[0, :OUT_DIM].set(b2)
    y2 = pl.pallas_call(
        _m2_body,
        grid=(GRID_M,),
        in_specs=[
            _row_spec(HALF),
            _row_spec(HALF),
            _row_spec(OUT_PAD),
            _row_spec(OUT_PAD),
            _full_spec(HID, OUT_PAD),
            _full_spec(1, HID),
        ],
        out_specs=_row_spec(OUT_PAD),
        out_shape=jax.ShapeDtypeStruct((N, OUT_PAD), jnp.float32),
    )(a0, a1, d0, d1, w2p, b1.reshape(1, HID))

    accw2 = _sc_scatter8(src_2, dst_2, y2)
    c0, c1 = accw2[:N], accw2[N:]

    out8 = pl.pallas_call(
        _m3_body,
        grid=(GRID_M,),
        in_specs=[
            _row_spec(OUT_PAD),
            _row_spec(OUT_PAD),
            _row_spec(OUT_PAD),
            _row_spec(OUT_PAD),
            _row_spec(OUT_PAD),
            _full_spec(1, OUT_PAD),
        ],
        out_specs=_row_spec(OUT_PAD),
        out_shape=jax.ShapeDtypeStruct((N, OUT_PAD), jnp.float32),
    )(c0, c1, y2, d0, d1, b2p)
    return out8[:, :OUT_DIM]


# speedup vs baseline: 10.7714x; 10.7714x over previous
import functools

import jax
import jax.numpy as jnp
from jax import lax
from jax.experimental import pallas as pl
from jax.experimental.pallas import tpu as pltpu
from jax.experimental.pallas import tpu_sc as plsc

N = 50000
E = 800000
IN_DIM = 1433
HID = 64
OUT_DIM = 7

NC = 2
NS = 16
NW = NC * NS
BN = 2176
NP = 23 * BN
GRID_N = NP // BN
N_TILE = NP // NS

CK = 4000
NCK = E // CK
NG = CK // 16
ECQ = E // 4

DCH = 125
DNCH = E // NS // DCH

_mesh = plsc.VectorSubcoreMesh(core_axis_name="c", subcore_axis_name="s")
_sc_params = pltpu.CompilerParams(use_tc_tiling_on_sc=False, needs_layout_passes=False)


@functools.partial(
    pl.kernel,
    out_type=jax.ShapeDtypeStruct((2 * NP,), jnp.float32),
    scratch_types=[
        pltpu.VMEM((DNCH, DCH), jnp.int32),
        pltpu.VMEM((DCH,), jnp.float32),
        pltpu.VMEM_SHARED((NP,), jnp.float32),
    ],
    mesh=_mesh,
    compiler_params=_sc_params,
)
def _sc_deg(dst_hbm, ones_hbm, onesn_hbm, out_hbm, dst_v, ones_v, acc):
    c = lax.axis_index("c")
    s = lax.axis_index("s")
    pltpu.sync_copy(dst_hbm.at[pl.ds(s * DNCH, DNCH)], dst_v)
    pltpu.sync_copy(ones_hbm, ones_v)
    pltpu.sync_copy(onesn_hbm.at[pl.ds(s * N_TILE, N_TILE)],
                    acc.at[pl.ds(s * N_TILE, N_TILE)])
    plsc.subcore_barrier()

    def body(j, carry):
        pltpu.sync_copy(ones_v, acc.at[dst_v.at[j]], add=True)
        return carry

    lax.fori_loop(0, DNCH, body, 0)
    plsc.subcore_barrier()
    pltpu.sync_copy(acc.at[pl.ds(s * N_TILE, N_TILE)],
                    out_hbm.at[pl.ds(c * NP + s * N_TILE, N_TILE)])


def _col_sweep(col_in, col_acc, src_hbm, dst_hbm, src_ch, dst_ch,
               e_base, n_chunks):
    def chunk(ch, carry):
        off = e_base + ch * CK
        pltpu.sync_copy(src_hbm.at[pl.ds(off, CK)], src_ch)
        pltpu.sync_copy(dst_hbm.at[pl.ds(off, CK)], dst_ch)

        def grp(g, c2):
            b = g * 16
            sv = src_ch[pl.ds(b, 16)]
            dv = dst_ch[pl.ds(b, 16)]
            vals = plsc.load_gather(col_in, [sv])
            plsc.addupdate_scatter(col_acc, [dv], vals)
            return c2

        lax.fori_loop(0, NG, grp, 0, unroll=8)
        return carry

    lax.fori_loop(0, n_chunks, chunk, 0)


@functools.partial(
    pl.kernel,
    out_type=jax.ShapeDtypeStruct((HID, NP), jnp.float32),
    scratch_types=[
        pltpu.VMEM((NP,), jnp.float32),
        pltpu.VMEM((NP,), jnp.float32),
        pltpu.VMEM((CK,), jnp.int32),
        pltpu.VMEM((CK,), jnp.int32),
    ],
    mesh=_mesh,
    compiler_params=_sc_params,
)
def _sc_l1(src_hbm, dst_hbm, yt_hbm, out_hbm,
           col_in, col_acc, src_ch, dst_ch):
    c = lax.axis_index("c")
    s = lax.axis_index("s")
    wid = s * NC + c
    for p in range(2):
        col = wid + 32 * p
        pltpu.sync_copy(yt_hbm.at[col], col_in)
        pltpu.sync_copy(yt_hbm.at[col], col_acc)
        _col_sweep(col_in, col_acc, src_hbm, dst_hbm, src_ch, dst_ch,
                   0, NCK)
        pltpu.sync_copy(col_acc, out_hbm.at[col])


@functools.partial(
    pl.kernel,
    out_type=jax.ShapeDtypeStruct((NW, NP), jnp.float32),
    scratch_types=[
        pltpu.VMEM((NP,), jnp.float32),
        pltpu.VMEM((NP,), jnp.float32),
        pltpu.VMEM((CK,), jnp.int32),
        pltpu.VMEM((CK,), jnp.int32),
    ],
    mesh=_mesh,
    compiler_params=_sc_params,
)
def _sc_l2(src_hbm, dst_hbm, y2t_hbm, zeros_hbm, out_hbm,
           col_in, col_acc, src_ch, dst_ch):
    c = lax.axis_index("c")
    s = lax.axis_index("s")
    wid = s * NC + c
    col = lax.rem(wid, 8)
    q = lax.div(wid, 8)
    pltpu.sync_copy(y2t_hbm.at[col], col_in)

    @pl.when(q == 0)
    def _():
        pltpu.sync_copy(y2t_hbm.at[col], col_acc)

    @pl.when(q > 0)
    def _():
        pltpu.sync_copy(zeros_hbm, col_acc)

    _col_sweep(col_in, col_acc, src_hbm, dst_hbm, src_ch, dst_ch,
               q * ECQ, ECQ // CK)
    pltpu.sync_copy(col_acc, out_hbm.at[wid])


def _m1_body(x_ref, w_ref, d_ref, yt_ref):
    dis = lax.rsqrt(d_ref[...])
    xwt = lax.dot_general(w_ref[...], x_ref[...],
                          (((0,), (1,)), ((), ())),
                          preferred_element_type=jnp.float32)
    yt_ref[...] = dis * xwt


def _m2_body(a_ref, d_ref, w2t_ref, b1_ref, y2t_ref):
    dis = lax.rsqrt(d_ref[...])
    h = jnp.maximum(dis * a_ref[...] + b1_ref[...], 0.0)
    y2t = lax.dot_general(w2t_ref[...], h,
                          (((1,), (0,)), ((), ())),
                          preferred_element_type=jnp.float32)
    y2t_ref[...] = dis * y2t


def _m3_body(p_ref, d_ref, b2_ref, o_ref):
    dis = lax.rsqrt(d_ref[...])
    p = p_ref[...]
    tot = p[0:8] + p[8:16] + p[16:24] + p[24:32]
    o_ref[...] = dis * tot + b2_ref[...]


def kernel(x, edge_index, W1, b1, W2, b2):
    ei = edge_index.astype(jnp.int32)
    src = ei[0]
    dst = ei[1]
    dst_d = dst.reshape(E // DCH, DCH)
    ones_ch = jnp.ones((DCH,), jnp.float32)
    ones_n = jnp.ones((NP,), jnp.float32)
    zeros_n = jnp.zeros((NP,), jnp.float32)

    d2 = _sc_deg(dst_d, ones_ch, ones_n).reshape(2, NP)[:1]

    yt = pl.pallas_call(
        _m1_body,
        grid=(GRID_N,),
        in_specs=[
            pl.BlockSpec((BN, IN_DIM), lambda i: (i, 0)),
            pl.BlockSpec((IN_DIM, HID), lambda i: (0, 0)),
            pl.BlockSpec((1, BN), lambda i: (0, i)),
        ],
        out_specs=pl.BlockSpec((HID, BN), lambda i: (0, i)),
        out_shape=jax.ShapeDtypeStruct((HID, NP), jnp.float32),
    )(x, W1, d2)

    at = _sc_l1(src, dst, yt)

    w2t = jnp.zeros((HID, 16), jnp.float32).at[:, :OUT_DIM].set(W2).T
    y2t = pl.pallas_call(
        _m2_body,
        grid=(GRID_N,),
        in_specs=[
            pl.BlockSpec((HID, BN), lambda i: (0, i)),
            pl.BlockSpec((1, BN), lambda i: (0, i)),
            pl.BlockSpec((16, HID), lambda i: (0, 0)),
            pl.BlockSpec((HID, 1), lambda i: (0, 0)),
        ],
        out_specs=pl.BlockSpec((16, BN), lambda i: (0, i)),
        out_shape=jax.ShapeDtypeStruct((16, NP), jnp.float32),
    )(at, d2, w2t, b1.reshape(HID, 1))

    pt = _sc_l2(src, dst, y2t, zeros_n)

    b2p = jnp.zeros((8, 1), jnp.float32).at[:OUT_DIM, 0].set(b2)
    outt = pl.pallas_call(
        _m3_body,
        grid=(GRID_N,),
        in_specs=[
            pl.BlockSpec((NW, BN), lambda i: (0, i)),
            pl.BlockSpec((1, BN), lambda i: (0, i)),
            pl.BlockSpec((8, 1), lambda i: (0, 0)),
        ],
        out_specs=pl.BlockSpec((8, BN), lambda i: (0, i)),
        out_shape=jax.ShapeDtypeStruct((8, NP), jnp.float32),
    )(pt, d2, b2p)
    return outt[:OUT_DIM, :N].T


# speedup vs baseline: 15.2909x; 1.4196x over previous
import functools

import jax
import jax.numpy as jnp
from jax import lax
from jax.experimental import pallas as pl
from jax.experimental.pallas import tpu as pltpu
from jax.experimental.pallas import tpu_sc as plsc

N = 50000
E = 800000
IN_DIM = 1433
HID = 64
OUT_DIM = 7

NC = 2
NS = 16
NW = NC * NS
BN = 2176
NP = 23 * BN
GRID_N = NP // BN
N_TILE = NP // NS

CK = 4000
NCK = E // CK
NG = CK // 16
ECQ = E // 4

DCH = 125
DNCH = E // NS // DCH

_mesh = plsc.VectorSubcoreMesh(core_axis_name="c", subcore_axis_name="s")
_sc_params = pltpu.CompilerParams(use_tc_tiling_on_sc=False, needs_layout_passes=False)


@functools.partial(
    pl.kernel,
    out_type=jax.ShapeDtypeStruct((2 * NP,), jnp.float32),
    scratch_types=[
        pltpu.VMEM((DNCH, DCH), jnp.int32),
        pltpu.VMEM((DCH,), jnp.float32),
        pltpu.VMEM_SHARED((NP,), jnp.float32),
    ],
    mesh=_mesh,
    compiler_params=_sc_params,
)
def _sc_deg(dst_hbm, ones_hbm, onesn_hbm, out_hbm, dst_v, ones_v, acc):
    c = lax.axis_index("c")
    s = lax.axis_index("s")
    pltpu.sync_copy(dst_hbm.at[pl.ds(s * DNCH, DNCH)], dst_v)
    pltpu.sync_copy(ones_hbm, ones_v)
    pltpu.sync_copy(onesn_hbm.at[pl.ds(s * N_TILE, N_TILE)],
                    acc.at[pl.ds(s * N_TILE, N_TILE)])
    plsc.subcore_barrier()

    def body(j, carry):
        pltpu.sync_copy(ones_v, acc.at[dst_v.at[j]], add=True)
        return carry

    lax.fori_loop(0, DNCH, body, 0)
    plsc.subcore_barrier()
    pltpu.sync_copy(acc.at[pl.ds(s * N_TILE, N_TILE)],
                    out_hbm.at[pl.ds(c * NP + s * N_TILE, N_TILE)])


def _col_sweep(col_in, col_acc, src_hbm, dst_hbm, bufs, sems,
               e_base, n_chunks):
    s0, d0, s1, d1 = bufs
    sem0, sem1 = sems

    def start(ch, sb, db, sem):
        off = e_base + ch * CK
        pltpu.async_copy(src_hbm.at[pl.ds(off, CK)], sb, sem)
        pltpu.async_copy(dst_hbm.at[pl.ds(off, CK)], db, sem)

    def drain(sb, db, sem):
        pltpu.make_async_copy(src_hbm.at[pl.ds(0, CK)], sb, sem).wait()
        pltpu.make_async_copy(dst_hbm.at[pl.ds(0, CK)], db, sem).wait()

    def compute(sb, db):
        def grp(g, c2):
            b = g * 16
            sv = sb[pl.ds(b, 16)]
            dv = db[pl.ds(b, 16)]
            vals = plsc.load_gather(col_in, [sv])
            plsc.addupdate_scatter(col_acc, [dv], vals)
            return c2

        lax.fori_loop(0, NG, grp, 0, unroll=8)

    start(0, s0, d0, sem0)
    n_half = n_chunks // 2

    def body(i, carry):
        ch = 2 * i
        start(ch + 1, s1, d1, sem1)
        drain(s0, d0, sem0)
        compute(s0, d0)

        @pl.when(i < n_half - 1)
        def _():
            start(ch + 2, s0, d0, sem0)

        drain(s1, d1, sem1)
        compute(s1, d1)
        return carry

    lax.fori_loop(0, n_half, body, 0)


@functools.partial(
    pl.kernel,
    out_type=jax.ShapeDtypeStruct((HID, NP), jnp.float32),
    scratch_types=[
        pltpu.VMEM((NP,), jnp.float32),
        pltpu.VMEM((NP,), jnp.float32),
        pltpu.VMEM((CK,), jnp.int32),
        pltpu.VMEM((CK,), jnp.int32),
        pltpu.VMEM((CK,), jnp.int32),
        pltpu.VMEM((CK,), jnp.int32),
        pltpu.SemaphoreType.DMA,
        pltpu.SemaphoreType.DMA,
    ],
    mesh=_mesh,
    compiler_params=_sc_params,
)
def _sc_l1(src_hbm, dst_hbm, yt_hbm, out_hbm,
           col_in, col_acc, s0, d0, s1, d1, sem0, sem1):
    c = lax.axis_index("c")
    s = lax.axis_index("s")
    wid = s * NC + c
    for p in range(2):
        col = wid + 32 * p
        pltpu.sync_copy(yt_hbm.at[col], col_in)
        pltpu.sync_copy(yt_hbm.at[col], col_acc)
        _col_sweep(col_in, col_acc, src_hbm, dst_hbm,
                   (s0, d0, s1, d1), (sem0, sem1), 0, NCK)
        pltpu.sync_copy(col_acc, out_hbm.at[col])


@functools.partial(
    pl.kernel,
    out_type=jax.ShapeDtypeStruct((NW, NP), jnp.float32),
    scratch_types=[
        pltpu.VMEM((NP,), jnp.float32),
        pltpu.VMEM((NP,), jnp.float32),
        pltpu.VMEM((CK,), jnp.int32),
        pltpu.VMEM((CK,), jnp.int32),
        pltpu.VMEM((CK,), jnp.int32),
        pltpu.VMEM((CK,), jnp.int32),
        pltpu.SemaphoreType.DMA,
        pltpu.SemaphoreType.DMA,
    ],
    mesh=_mesh,
    compiler_params=_sc_params,
)
def _sc_l2(src_hbm, dst_hbm, y2t_hbm, zeros_hbm, out_hbm,
           col_in, col_acc, s0, d0, s1, d1, sem0, sem1):
    c = lax.axis_index("c")
    s = lax.axis_index("s")
    wid = s * NC + c
    col = lax.rem(wid, 8)
    q = lax.div(wid, 8)
    pltpu.sync_copy(y2t_hbm.at[col], col_in)

    @pl.when(q == 0)
    def _():
        pltpu.sync_copy(y2t_hbm.at[col], col_acc)

    @pl.when(q > 0)
    def _():
        pltpu.sync_copy(zeros_hbm, col_acc)

    _col_sweep(col_in, col_acc, src_hbm, dst_hbm,
               (s0, d0, s1, d1), (sem0, sem1), q * ECQ, ECQ // CK)
    pltpu.sync_copy(col_acc, out_hbm.at[wid])


def _m1_body(x_ref, w_ref, d_ref, yt_ref):
    dis = lax.rsqrt(d_ref[...])
    xwt = lax.dot_general(w_ref[...], x_ref[...],
                          (((0,), (1,)), ((), ())),
                          preferred_element_type=jnp.float32)
    yt_ref[...] = dis * xwt


def _m2_body(a_ref, d_ref, w2t_ref, b1_ref, y2t_ref):
    dis = lax.rsqrt(d_ref[...])
    h = jnp.maximum(dis * a_ref[...] + b1_ref[...], 0.0)
    y2t = lax.dot_general(w2t_ref[...], h,
                          (((1,), (0,)), ((), ())),
                          preferred_element_type=jnp.float32)
    y2t_ref[...] = dis * y2t


def _m3_body(p_ref, d_ref, b2_ref, o_ref):
    dis = lax.rsqrt(d_ref[...])
    p = p_ref[...]
    tot = p[0:8] + p[8:16] + p[16:24] + p[24:32]
    o_ref[...] = dis * tot + b2_ref[...]


def kernel(x, edge_index, W1, b1, W2, b2):
    ei = edge_index.astype(jnp.int32)
    src = ei[0]
    dst = ei[1]
    dst_d = dst.reshape(E // DCH, DCH)
    ones_ch = jnp.ones((DCH,), jnp.float32)
    ones_n = jnp.ones((NP,), jnp.float32)
    zeros_n = jnp.zeros((NP,), jnp.float32)

    d2 = _sc_deg(dst_d, ones_ch, ones_n).reshape(2, NP)[:1]

    yt = pl.pallas_call(
        _m1_body,
        grid=(GRID_N,),
        in_specs=[
            pl.BlockSpec((BN, IN_DIM), lambda i: (i, 0)),
            pl.BlockSpec((IN_DIM, HID), lambda i: (0, 0)),
            pl.BlockSpec((1, BN), lambda i: (0, i)),
        ],
        out_specs=pl.BlockSpec((HID, BN), lambda i: (0, i)),
        out_shape=jax.ShapeDtypeStruct((HID, NP), jnp.float32),
    )(x, W1, d2)

    at = _sc_l1(src, dst, yt)

    w2t = jnp.zeros((HID, 16), jnp.float32).at[:, :OUT_DIM].set(W2).T
    y2t = pl.pallas_call(
        _m2_body,
        grid=(GRID_N,),
        in_specs=[
            pl.BlockSpec((HID, BN), lambda i: (0, i)),
            pl.BlockSpec((1, BN), lambda i: (0, i)),
            pl.BlockSpec((16, HID), lambda i: (0, 0)),
            pl.BlockSpec((HID, 1), lambda i: (0, 0)),
        ],
        out_specs=pl.BlockSpec((16, BN), lambda i: (0, i)),
        out_shape=jax.ShapeDtypeStruct((16, NP), jnp.float32),
    )(at, d2, w2t, b1.reshape(HID, 1))

    pt = _sc_l2(src, dst, y2t, zeros_n)

    b2p = jnp.zeros((8, 1), jnp.float32).at[:OUT_DIM, 0].set(b2)
    outt = pl.pallas_call(
        _m3_body,
        grid=(GRID_N,),
        in_specs=[
            pl.BlockSpec((NW, BN), lambda i: (0, i)),
            pl.BlockSpec((1, BN), lambda i: (0, i)),
            pl.BlockSpec((8, 1), lambda i: (0, 0)),
        ],
        out_specs=pl.BlockSpec((8, BN), lambda i: (0, i)),
        out_shape=jax.ShapeDtypeStruct((8, NP), jnp.float32),
    )(pt, d2, b2p)
    return outt[:OUT_DIM, :N].T
